# baseline (speedup 1.0000x reference)
import functools
import math

import jax
import jax.numpy as jnp
from jax import lax
from jax.experimental import pallas as pl
from jax.experimental.pallas import tpu as pltpu
from jax.experimental.pallas import tpu_sc as plsc

_SPARSITY = 0.9
_NC = 2
_NS = 16
_NW = _NC * _NS
_LANES = 16
_ABS_MASK = 0x7FFFFFFF
_SENTINEL = 0x7FFFFFFF


def _make(n_rows, row_len, r_chunk, unroll=32, interpret=False):
  assert n_rows % (_NW * r_chunk) == 0
  assert row_len % (2 * _LANES * unroll) == 0
  rows_per_w = n_rows // _NW
  n_chunks = rows_per_w // r_chunk
  n_sparse = math.floor((1.0 - _SPARSITY) * row_len)
  k_rank = row_len - n_sparse - 1
  vregs_per_row = row_len // _LANES
  hvregs_per_row = row_len // (2 * _LANES)
  chunk_elems = r_chunk * row_len

  mesh = plsc.VectorSubcoreMesh(
      core_axis_name="c", subcore_axis_name="s",
      num_cores=_NC, num_subcores=_NS)

  @functools.partial(
      pl.kernel,
      out_type=jax.ShapeDtypeStruct((n_rows,), jnp.int32),
      mesh=mesh,
      scratch_types=[
          pltpu.VMEM((chunk_elems,), jnp.float32),
          pltpu.VMEM((chunk_elems,), jnp.float32),
          pltpu.VMEM((chunk_elems // 2,), jnp.int32),
          pltpu.VMEM((r_chunk + _LANES,), jnp.int32),
          pltpu.SemaphoreType.DMA,
          pltpu.SemaphoreType.DMA,
      ],
      compiler_params=pltpu.CompilerParams(needs_layout_passes=False),
      interpret=interpret,
  )
  def sc_kernel(x_hbm, t_hbm, xbuf0, xbuf1, hbuf, thrbuf, sem0, sem1):
    wid = lax.axis_index("s") * _NC + lax.axis_index("c")
    wbase = wid * rows_per_w * row_len
    kv = jnp.full((_LANES,), k_rank, jnp.int32)
    ones_v = jnp.full((_LANES,), 1, jnp.int32)
    lane0 = lax.iota(jnp.int32, _LANES) == 0
    xbufs = (xbuf0, xbuf1)
    sems = (sem0, sem1)

    def in_slice(ci):
      return x_hbm.at[pl.ds(wbase + ci * chunk_elems, chunk_elems)]

    pltpu.async_copy(in_slice(0), xbufs[0], sems[0])
    pltpu.async_copy(in_slice(1), xbufs[1], sems[1])

    def process_chunk(ci, xbuf):
      def pk_body(j, _):
        for u in range(unroll // 2):
          o2 = (j * (unroll // 2) + u) * 2 * _LANES
          a0 = (plsc.bitcast(xbuf[pl.ds(o2, _LANES)], jnp.int32)
                & _ABS_MASK) >> 16
          a1 = (plsc.bitcast(xbuf[pl.ds(o2 + _LANES, _LANES)], jnp.int32)
                & _ABS_MASK) >> 16
          packed = plsc.pack(
              a0, a1, format=plsc.PackFormat.INTERLEAVED,
              preferred_element_type=jnp.int16)
          hbuf[pl.ds(o2 // 2, _LANES)] = plsc.bitcast(packed, jnp.int32)
        return 0
      lax.fori_loop(0, chunk_elems // (_LANES * unroll), pk_body, 0,
                    unroll=False)

      def row_body(r, _):
        rbase = r * row_len

        def hcount(cand_v):
          c16 = plsc.pack(cand_v, cand_v,
                          format=plsc.PackFormat.INTERLEAVED,
                          preferred_element_type=jnp.int16)

          def cnt_body(j, acc):
            for u in range(unroll):
              v = hbuf[pl.ds(rbase // 2 + (j * unroll + u) * _LANES,
                             _LANES)]
              m = plsc.bitcast(v, jnp.int16) < c16
              acc = acc + plsc.all_reduce_population_count(m, reduce=2)
            return acc
          return lax.fori_loop(0, hvregs_per_row // unroll, cnt_body,
                               jnp.zeros((_LANES,), jnp.int32),
                               unroll=False)

        def hbit_body(i, res_h):
          cand = res_h | jnp.left_shift(ones_v, 14 - i)
          return jnp.where(hcount(cand) <= kv, cand, res_h)
        res_h = lax.fori_loop(0, 15, hbit_body,
                              jnp.zeros((_LANES,), jnp.int32),
                              unroll=False)

        hv = res_h

        def lp_body(j, c0acc):
          for u in range(unroll // 2):
            o2 = rbase + (j * (unroll // 2) + u) * 2 * _LANES

            def masked_lo(a):
              hi = a >> 16
              lo = (a & 0xFFFF) ^ 0x8000
              return hi, jnp.where(hi == hv, lo, 0x7FFF)
            hi0, lo0 = masked_lo(
                plsc.bitcast(xbuf[pl.ds(o2, _LANES)], jnp.int32)
                & _ABS_MASK)
            hi1, lo1 = masked_lo(
                plsc.bitcast(xbuf[pl.ds(o2 + _LANES, _LANES)], jnp.int32)
                & _ABS_MASK)
            c0acc = c0acc + plsc.all_reduce_population_count(hi0 < hv)
            c0acc = c0acc + plsc.all_reduce_population_count(hi1 < hv)
            packed = plsc.pack(lo0, lo1,
                               format=plsc.PackFormat.INTERLEAVED,
                               preferred_element_type=jnp.int16)
            hbuf[pl.ds(o2 // 2, _LANES)] = plsc.bitcast(packed, jnp.int32)
          return c0acc
        c0v = lax.fori_loop(0, vregs_per_row // unroll, lp_body,
                            jnp.zeros((_LANES,), jnp.int32),
                            unroll=False)

        def lbit_body(i, res):
          cand = res | jnp.left_shift(ones_v, 15 - i)
          cnt = hcount((cand ^ 0x8000) & 0xFFFF)
          return jnp.where(c0v + cnt <= kv, cand, res)
        res_l = lax.fori_loop(0, 16, lbit_body,
                              jnp.zeros((_LANES,), jnp.int32),
                              unroll=False)
        res = (hv << 16) | res_l

        plsc.store_compressed(thrbuf.at[pl.ds(r, _LANES)], res,
                              mask=lane0)
        return 0

      lax.fori_loop(0, r_chunk, row_body, 0, unroll=False)
      trow = wid * rows_per_w + ci * r_chunk
      pltpu.sync_copy(thrbuf.at[pl.ds(0, r_chunk)],
                      t_hbm.at[pl.ds(trow, r_chunk)])

    def chunk_pair(cp, _):
      for b in range(2):
        ci = cp * 2 + b
        pltpu.make_async_copy(in_slice(ci), xbufs[b], sems[b]).wait()
        process_chunk(ci, xbufs[b])
        ci2 = jnp.minimum(ci + 2, n_chunks - 1)
        pltpu.async_copy(in_slice(ci2), xbufs[b], sems[b])
      return 0

    lax.fori_loop(0, n_chunks // 2, chunk_pair, 0, unroll=False)
    for b in range(2):
      pltpu.make_async_copy(in_slice(0), xbufs[b], sems[b]).wait()

  return sc_kernel


def _tc_mask_kernel(x_ref, t_ref, o_ref):
  xv = x_ref[...]
  xb = lax.bitcast_convert_type(xv, jnp.int32)
  o_ref[...] = jnp.where((xb & _ABS_MASK) >= t_ref[...], xv,
                         jnp.float32(0))


def kernel(x):
  shape = x.shape
  row_len = shape[-1]
  n_rows = x.size // row_len
  sc_kernel = _make(n_rows, row_len, r_chunk=16)
  thr_bits = sc_kernel(x.reshape(-1))
  b, s = shape[0], shape[1]
  rb = 512
  out = pl.pallas_call(
      _tc_mask_kernel,
      out_shape=jax.ShapeDtypeStruct(shape, jnp.float32),
      grid=(b, s // rb),
      in_specs=[
          pl.BlockSpec((1, rb, row_len), lambda i, j: (i, j, 0)),
          pl.BlockSpec((1, rb, 1), lambda i, j: (i, j, 0)),
      ],
      out_specs=pl.BlockSpec((1, rb, row_len), lambda i, j: (i, j, 0)),
  )(x, thr_bits.reshape(b, s, 1))
  return out


# speedup vs baseline: 1.0878x; 1.0878x over previous
import functools
import math

import jax
import jax.numpy as jnp
from jax import lax
from jax.experimental import pallas as pl
from jax.experimental.pallas import tpu as pltpu
from jax.experimental.pallas import tpu_sc as plsc

_SPARSITY = 0.9
_NC = 2
_NS = 16
_NW = _NC * _NS
_LANES = 16
_ABS_MASK = 0x7FFFFFFF


def _make(n_rows, row_len, r_chunk, unroll=16, interpret=False):
  assert n_rows % (_NW * r_chunk) == 0
  assert row_len % (2 * _LANES * unroll) == 0
  rows_per_w = n_rows // _NW
  n_chunks = rows_per_w // r_chunk
  n_sparse = math.floor((1.0 - _SPARSITY) * row_len)
  k_rank = row_len - n_sparse - 1
  vregs_per_row = row_len // _LANES
  hvregs_per_row = row_len // (2 * _LANES)
  chunk_elems = r_chunk * row_len

  mesh = plsc.VectorSubcoreMesh(
      core_axis_name="c", subcore_axis_name="s",
      num_cores=_NC, num_subcores=_NS)

  @functools.partial(
      pl.kernel,
      out_type=jax.ShapeDtypeStruct((n_rows,), jnp.int32),
      mesh=mesh,
      scratch_types=[
          pltpu.VMEM((chunk_elems,), jnp.float32),
          pltpu.VMEM((chunk_elems,), jnp.float32),
          pltpu.VMEM((chunk_elems // 2,), jnp.int32),
          pltpu.VMEM((r_chunk + _LANES,), jnp.int32),
          pltpu.SemaphoreType.DMA,
          pltpu.SemaphoreType.DMA,
      ],
      compiler_params=pltpu.CompilerParams(needs_layout_passes=False),
      interpret=interpret,
  )
  def sc_kernel(x_hbm, t_hbm, xbuf0, xbuf1, hbuf, thrbuf, sem0, sem1):
    wid = lax.axis_index("s") * _NC + lax.axis_index("c")
    wbase = wid * rows_per_w * row_len
    kv = jnp.full((_LANES,), k_rank, jnp.int32)
    ones_v = jnp.full((_LANES,), 1, jnp.int32)
    lane0 = lax.iota(jnp.int32, _LANES) == 0
    xbufs = (xbuf0, xbuf1)
    sems = (sem0, sem1)

    def in_slice(ci):
      return x_hbm.at[pl.ds(wbase + ci * chunk_elems, chunk_elems)]

    pltpu.async_copy(in_slice(0), xbufs[0], sems[0])
    pltpu.async_copy(in_slice(1), xbufs[1], sems[1])

    def process_chunk(ci, xbuf):
      def pk_body(j, _):
        for u in range(unroll // 2):
          o2 = (j * (unroll // 2) + u) * 2 * _LANES
          a0 = (plsc.bitcast(xbuf[pl.ds(o2, _LANES)], jnp.int32)
                & _ABS_MASK) >> 16
          a1 = (plsc.bitcast(xbuf[pl.ds(o2 + _LANES, _LANES)], jnp.int32)
                & _ABS_MASK) >> 16
          packed = plsc.pack(
              a0, a1, format=plsc.PackFormat.INTERLEAVED,
              preferred_element_type=jnp.int16)
          hbuf[pl.ds(o2 // 2, _LANES)] = plsc.bitcast(packed, jnp.int32)
        return 0
      lax.fori_loop(0, chunk_elems // (_LANES * unroll), pk_body, 0,
                    unroll=False)

      def row_body(r, _):
        rbase = r * row_len

        def hcount(cand_v):
          c16 = plsc.pack(cand_v, cand_v,
                          format=plsc.PackFormat.INTERLEAVED,
                          preferred_element_type=jnp.int16)

          def cnt_body(j, acc):
            for u in range(unroll):
              v = hbuf[pl.ds(rbase // 2 + (j * unroll + u) * _LANES,
                             _LANES)]
              m = plsc.bitcast(v, jnp.int16) < c16
              acc = acc + plsc.all_reduce_population_count(m, reduce=2)
            return acc
          return lax.fori_loop(0, hvregs_per_row // unroll, cnt_body,
                               jnp.zeros((_LANES,), jnp.int32),
                               unroll=False)

        def hbit_body(i, res_h):
          cand = res_h | jnp.left_shift(ones_v, 14 - i)
          return jnp.where(hcount(cand) <= kv, cand, res_h)
        res_h = lax.fori_loop(0, 15, hbit_body,
                              jnp.zeros((_LANES,), jnp.int32),
                              unroll=False)

        hv = res_h

        def lp_body(j, c0acc):
          for u in range(unroll // 2):
            o2 = rbase + (j * (unroll // 2) + u) * 2 * _LANES

            def masked_lo(a):
              hi = a >> 16
              lo = (a & 0xFFFF) ^ 0x8000
              return hi, jnp.where(hi == hv, lo, 0x7FFF)
            hi0, lo0 = masked_lo(
                plsc.bitcast(xbuf[pl.ds(o2, _LANES)], jnp.int32)
                & _ABS_MASK)
            hi1, lo1 = masked_lo(
                plsc.bitcast(xbuf[pl.ds(o2 + _LANES, _LANES)], jnp.int32)
                & _ABS_MASK)
            c0acc = c0acc + plsc.all_reduce_population_count(hi0 < hv)
            c0acc = c0acc + plsc.all_reduce_population_count(hi1 < hv)
            packed = plsc.pack(lo0, lo1,
                               format=plsc.PackFormat.INTERLEAVED,
                               preferred_element_type=jnp.int16)
            hbuf[pl.ds(o2 // 2, _LANES)] = plsc.bitcast(packed, jnp.int32)
          return c0acc
        c0v = lax.fori_loop(0, vregs_per_row // unroll, lp_body,
                            jnp.zeros((_LANES,), jnp.int32),
                            unroll=False)

        def lbit_body(i, res):
          cand = res | jnp.left_shift(ones_v, 15 - i)
          cnt = hcount((cand ^ 0x8000) & 0xFFFF)
          return jnp.where(c0v + cnt <= kv, cand, res)
        res_l = lax.fori_loop(0, 16, lbit_body,
                              jnp.zeros((_LANES,), jnp.int32),
                              unroll=False)
        res = (hv << 16) | res_l

        plsc.store_compressed(thrbuf.at[pl.ds(r, _LANES)], res,
                              mask=lane0)
        return 0

      lax.fori_loop(0, r_chunk, row_body, 0, unroll=False)
      trow = wid * rows_per_w + ci * r_chunk
      pltpu.sync_copy(thrbuf.at[pl.ds(0, r_chunk)],
                      t_hbm.at[pl.ds(trow, r_chunk)])

    def chunk_pair(cp, _):
      for b in range(2):
        ci = cp * 2 + b
        pltpu.make_async_copy(in_slice(ci), xbufs[b], sems[b]).wait()
        process_chunk(ci, xbufs[b])
        ci2 = jnp.minimum(ci + 2, n_chunks - 1)
        pltpu.async_copy(in_slice(ci2), xbufs[b], sems[b])
      return 0

    lax.fori_loop(0, n_chunks // 2, chunk_pair, 0, unroll=False)
    for b in range(2):
      pltpu.make_async_copy(in_slice(0), xbufs[b], sems[b]).wait()

  return sc_kernel


def _tc_mask_kernel(x_ref, t_ref, o_ref):
  xv = x_ref[...]
  xb = lax.bitcast_convert_type(xv, jnp.int32)
  o_ref[...] = jnp.where((xb & _ABS_MASK) >= t_ref[...], xv,
                         jnp.float32(0))


def kernel(x):
  shape = x.shape
  row_len = shape[-1]
  n_rows = x.size // row_len
  sc_kernel = _make(n_rows, row_len, r_chunk=16)
  thr_bits = sc_kernel(x.reshape(-1))
  b, s = shape[0], shape[1]
  rb = 512
  out = pl.pallas_call(
      _tc_mask_kernel,
      out_shape=jax.ShapeDtypeStruct(shape, jnp.float32),
      grid=(b, s // rb),
      in_specs=[
          pl.BlockSpec((1, rb, row_len), lambda i, j: (i, j, 0)),
          pl.BlockSpec((1, rb, 1), lambda i, j: (i, j, 0)),
      ],
      out_specs=pl.BlockSpec((1, rb, row_len), lambda i, j: (i, j, 0)),
  )(x, thr_bits.reshape(b, s, 1))
  return out
